# Initial kernel scaffold; baseline (speedup 1.0000x reference)
import jax
import jax.numpy as jnp
from jax.experimental import pallas as pl


def kernel(x, edge_index, edge_weight, edge_type, w1, root1, b1, w2, root2, b2):
    raise NotImplementedError("write your pallas kernel here")


# speedup vs baseline: 19.1696x; 19.1696x over previous
import dataclasses
import functools
import jax
import jax.numpy as jnp
from jax import lax
from jax.experimental import pallas as pl
from jax.experimental.pallas import tpu as pltpu
from jax.experimental.pallas import tpu_sc as plsc

NREL, NN, NE = 8, 10000, 320000
DIN, DHID, DOUT = 128, 128, 40
DP = 48
NC, NS = 2, 16
NW = NC * NS
SUB = 80
ROWS = 25
SUP = SUB * ROWS
EPW = NE // NW
EPS = NE // NS
NSEG = NN * NREL
SEG_SL = NSEG // NS
NZR = NN // NS
ZR = 125

_mesh = plsc.VectorSubcoreMesh(core_axis_name="c", subcore_axis_name="s",
                               num_cores=NC, num_subcores=NS)

_cp = pltpu.CompilerParams()
if "needs_layout_passes" in pltpu.CompilerParams.__dataclass_fields__:
    _cp = dataclasses.replace(_cp, needs_layout_passes=False)
if "use_tc_tiling_on_sc" in pltpu.CompilerParams.__dataclass_fields__:
    _cp = dataclasses.replace(_cp, use_tc_tiling_on_sc=False)


def _splat16(v):
    return jnp.broadcast_to(jnp.asarray(v, jnp.int32), (16,))


@functools.partial(
    pl.kernel,
    out_type=jax.ShapeDtypeStruct((NE,), jnp.float32),
    mesh=_mesh,
    compiler_params=_cp,
    scratch_types=[
        pltpu.VMEM((SEG_SL,), jnp.float32),
        pltpu.VMEM((SUP,), jnp.int32),
        pltpu.VMEM((SUP,), jnp.int32),
        pltpu.VMEM((SUP,), jnp.float32),
        pltpu.VMEM((SUP,), jnp.float32),
        pltpu.VMEM((ROWS, SUB), jnp.int32),
        pltpu.VMEM((SUB,), jnp.float32),
        pltpu.VMEM((NSEG,), jnp.float32),
        pltpu.VMEM_SHARED((NSEG,), jnp.float32),
    ],
)
def _sc_prep(dst_hbm, typ_hbm, w_hbm, s_hbm,
             zbuf, dbuf, tbuf, wbuf, sbuf, segbuf, ones, cntl, cnts):
    sid = lax.axis_index("s")
    cid = lax.axis_index("c")

    @pl.loop(0, SEG_SL, step=16)
    def _(i):
        zbuf[pl.ds(i, 16)] = jnp.zeros((16,), jnp.float32)

    @pl.loop(0, SUB, step=16)
    def _(i):
        ones[pl.ds(i, 16)] = jnp.ones((16,), jnp.float32)

    pltpu.sync_copy(zbuf, cnts.at[pl.ds(sid * SEG_SL, SEG_SL)])
    plsc.subcore_barrier()

    @pl.loop(0, EPS, step=SUP)
    def _(i):
        base = sid * EPS + i
        pltpu.sync_copy(dst_hbm.at[pl.ds(base, SUP)], dbuf)
        pltpu.sync_copy(typ_hbm.at[pl.ds(base, SUP)], tbuf)

        @pl.loop(0, ROWS)
        def _(r):
            @pl.loop(0, SUB, step=16)
            def _(j):
                segbuf[r, pl.ds(j, 16)] = (
                    dbuf[pl.ds(r * SUB + j, 16)] * NREL
                    + tbuf[pl.ds(r * SUB + j, 16)])
            pltpu.sync_copy(ones, cnts.at[segbuf.at[r]], add=True)

    plsc.subcore_barrier()
    pltpu.sync_copy(cnts, cntl)

    wid = sid * NC + cid

    @pl.loop(0, EPW, step=SUP)
    def _(i):
        base = wid * EPW + i
        pltpu.sync_copy(dst_hbm.at[pl.ds(base, SUP)], dbuf)
        pltpu.sync_copy(typ_hbm.at[pl.ds(base, SUP)], tbuf)
        pltpu.sync_copy(w_hbm.at[pl.ds(base, SUP)], wbuf)

        @pl.loop(0, SUP, step=16)
        def _(j):
            seg = dbuf[pl.ds(j, 16)] * NREL + tbuf[pl.ds(j, 16)]
            c = plsc.load_gather(cntl, [seg])
            sbuf[pl.ds(j, 16)] = wbuf[pl.ds(j, 16)] / jnp.maximum(c, 1.0)

        pltpu.sync_copy(sbuf, s_hbm.at[pl.ds(base, SUP)])


def _make_agg(d):
    @functools.partial(
        pl.kernel,
        out_type=jax.ShapeDtypeStruct((NC, NN, d), jnp.float32),
        mesh=_mesh,
        compiler_params=_cp,
        scratch_types=[
            pltpu.VMEM((ZR, d), jnp.float32),
            pltpu.VMEM((SUP,), jnp.int32),
            pltpu.VMEM((SUP,), jnp.int32),
            pltpu.VMEM((SUP,), jnp.float32),
            pltpu.VMEM((ROWS, SUB), jnp.int32),
            pltpu.VMEM((ROWS, SUB), jnp.int32),
            pltpu.VMEM((SUB, d), jnp.float32),
            pltpu.VMEM_SHARED((NN, d), jnp.float32),
        ],
    )
    def _agg(tab_hbm, src_hbm, typ_hbm, dst_hbm, s_hbm, out_hbm,
             zbuf, srcb, typb, sb, gidx, didx, rows, acc):
        sid = lax.axis_index("s")
        cid = lax.axis_index("c")
        wid = sid * NC + cid

        @pl.loop(0, ZR)
        def _(r):
            for k in range(d // 16):
                zbuf[r, pl.ds(k * 16, 16)] = jnp.zeros((16,), jnp.float32)

        for k in range(NZR // ZR):
            pltpu.sync_copy(zbuf, acc.at[pl.ds(sid * NZR + k * ZR, ZR)])
        plsc.subcore_barrier()

        @pl.loop(0, EPW, step=SUP)
        def _(i):
            base = wid * EPW + i
            pltpu.sync_copy(src_hbm.at[pl.ds(base, SUP)], srcb)
            pltpu.sync_copy(typ_hbm.at[pl.ds(base, SUP)], typb)
            pltpu.sync_copy(s_hbm.at[pl.ds(base, SUP)], sb)

            @pl.loop(0, ROWS)
            def _(r):
                pltpu.sync_copy(dst_hbm.at[pl.ds(base + r * SUB, SUB)],
                                didx.at[r])
                @pl.loop(0, SUB, step=16)
                def _(j):
                    gidx[r, pl.ds(j, 16)] = (
                        typb[pl.ds(r * SUB + j, 16)] * NN
                        + srcb[pl.ds(r * SUB + j, 16)])

                pltpu.sync_copy(tab_hbm.at[gidx.at[r]], rows)

                @pl.loop(0, SUB)
                def _(e):
                    spl = plsc.load_gather(sb, [_splat16(r * SUB + e)])
                    for k in range(d // 16):
                        sl = pl.ds(k * 16, 16)
                        rows[e, sl] = rows[e, sl] * spl

                pltpu.sync_copy(rows, acc.at[didx.at[r]], add=True)

        plsc.subcore_barrier()

        @pl.when(sid == 0)
        def _():
            pltpu.sync_copy(acc, out_hbm.at[cid])

    return _agg


_agg_hid = _make_agg(DHID)
_agg_out = _make_agg(DP)


BN = 400
NB = NN // BN

_DOT = functools.partial(jnp.dot, preferred_element_type=jnp.float32,
                         precision=lax.Precision.HIGHEST)


def _t1_body(x_ref, w_ref, r_ref, b_ref, tab_ref, base_ref):
    xb = x_ref[...]
    for r in range(NREL):
        tab_ref[r] = _DOT(xb, w_ref[r])
    base_ref[...] = _DOT(xb, r_ref[...]) + b_ref[...]


_t1 = pl.pallas_call(
    _t1_body,
    grid=(NB,),
    in_specs=[
        pl.BlockSpec((BN, DIN), lambda i: (i, 0)),
        pl.BlockSpec((NREL, DIN, DHID), lambda i: (0, 0, 0)),
        pl.BlockSpec((DIN, DHID), lambda i: (0, 0)),
        pl.BlockSpec((1, DHID), lambda i: (0, 0)),
    ],
    out_specs=[
        pl.BlockSpec((NREL, BN, DHID), lambda i: (0, i, 0)),
        pl.BlockSpec((BN, DHID), lambda i: (i, 0)),
    ],
    out_shape=[
        jax.ShapeDtypeStruct((NREL, NN, DHID), jnp.float32),
        jax.ShapeDtypeStruct((NN, DHID), jnp.float32),
    ],
)


def _t2_body(b1_ref, a1_ref, w_ref, r_ref, b_ref, tab_ref, base_ref):
    h = jnp.maximum(b1_ref[...] + a1_ref[0] + a1_ref[1], 0.0)
    for r in range(NREL):
        tab_ref[r] = _DOT(h, w_ref[r])
    base_ref[...] = _DOT(h, r_ref[...]) + b_ref[...]


_t2 = pl.pallas_call(
    _t2_body,
    grid=(NB,),
    in_specs=[
        pl.BlockSpec((BN, DHID), lambda i: (i, 0)),
        pl.BlockSpec((NC, BN, DHID), lambda i: (0, i, 0)),
        pl.BlockSpec((NREL, DHID, DP), lambda i: (0, 0, 0)),
        pl.BlockSpec((DHID, DP), lambda i: (0, 0)),
        pl.BlockSpec((1, DP), lambda i: (0, 0)),
    ],
    out_specs=[
        pl.BlockSpec((NREL, BN, DP), lambda i: (0, i, 0)),
        pl.BlockSpec((BN, DP), lambda i: (i, 0)),
    ],
    out_shape=[
        jax.ShapeDtypeStruct((NREL, NN, DP), jnp.float32),
        jax.ShapeDtypeStruct((NN, DP), jnp.float32),
    ],
)


def _t3_body(b2_ref, a2_ref, out_ref):
    z = b2_ref[...] + a2_ref[0] + a2_ref[1]
    mask = lax.broadcasted_iota(jnp.int32, (BN, DP), 1) < DOUT
    zm = jnp.where(mask, z, -1e30)
    m = jnp.max(zm, axis=1, keepdims=True)
    lse = jnp.log(jnp.sum(jnp.exp(zm - m), axis=1, keepdims=True))
    out_ref[...] = z - m - lse


_t3 = pl.pallas_call(
    _t3_body,
    grid=(NB,),
    in_specs=[
        pl.BlockSpec((BN, DP), lambda i: (i, 0)),
        pl.BlockSpec((NC, BN, DP), lambda i: (0, i, 0)),
    ],
    out_specs=pl.BlockSpec((BN, DP), lambda i: (i, 0)),
    out_shape=jax.ShapeDtypeStruct((NN, DP), jnp.float32),
)


@jax.jit
def _run(x, edge_index, edge_weight, edge_type, w1, root1, b1, w2, root2, b2):
    src = edge_index[0]
    dst = edge_index[1]
    et = edge_type

    s = _sc_prep(dst, et, edge_weight)
    tab1, base1 = _t1(x, w1, root1, b1.reshape(1, DHID))
    acc1 = _agg_hid(tab1.reshape(NREL * NN, DHID), src, et, dst, s)

    w2p = jnp.pad(w2, ((0, 0), (0, 0), (0, DP - DOUT)))
    root2p = jnp.pad(root2, ((0, 0), (0, DP - DOUT)))
    b2p = jnp.pad(b2, (0, DP - DOUT)).reshape(1, DP)
    tab2, base2 = _t2(base1, acc1, w2p, root2p, b2p)
    acc2 = _agg_out(tab2.reshape(NREL * NN, DP), src, et, dst, s)

    out = _t3(base2, acc2)
    return out[:, :DOUT]


def kernel(x, edge_index, edge_weight, edge_type, w1, root1, b1,
           w2, root2, b2):
    return _run(x, edge_index, edge_weight, edge_type, w1, root1, b1,
                w2, root2, b2)


# speedup vs baseline: 31.7952x; 1.6586x over previous
import dataclasses
import functools
import jax
import jax.numpy as jnp
from jax import lax
from jax.experimental import pallas as pl
from jax.experimental.pallas import tpu as pltpu
from jax.experimental.pallas import tpu_sc as plsc

NREL, NN, NE = 8, 10000, 320000
DIN, DHID, DOUT = 128, 128, 40
DP = 48
NC, NS = 2, 16
NW = NC * NS
SUB = 80
ROWS = 25
SUP = SUB * ROWS
EPW = NE // NW
EPS = NE // NS
NSEG = NN * NREL
SEG_SL = NSEG // NS
NZR = NN // NS
ZR = 125

_mesh = plsc.VectorSubcoreMesh(core_axis_name="c", subcore_axis_name="s",
                               num_cores=NC, num_subcores=NS)

_cp = pltpu.CompilerParams()
if "needs_layout_passes" in pltpu.CompilerParams.__dataclass_fields__:
    _cp = dataclasses.replace(_cp, needs_layout_passes=False)
if "use_tc_tiling_on_sc" in pltpu.CompilerParams.__dataclass_fields__:
    _cp = dataclasses.replace(_cp, use_tc_tiling_on_sc=False)


def _splat16(v):
    return jnp.broadcast_to(jnp.asarray(v, jnp.int32), (16,))


@functools.partial(
    pl.kernel,
    out_type=(jax.ShapeDtypeStruct((NE,), jnp.int32),
              jax.ShapeDtypeStruct((NE,), jnp.float32)),
    mesh=_mesh,
    compiler_params=_cp,
    scratch_types=[
        pltpu.VMEM((SEG_SL,), jnp.float32),
        pltpu.VMEM((SUP,), jnp.int32),
        pltpu.VMEM((SUP,), jnp.int32),
        pltpu.VMEM((SUP,), jnp.int32),
        pltpu.VMEM((SUP,), jnp.float32),
        pltpu.VMEM((SUP,), jnp.int32),
        pltpu.VMEM((SUP,), jnp.float32),
        pltpu.VMEM((ROWS, SUB), jnp.int32),
        pltpu.VMEM((SUB,), jnp.float32),
        pltpu.VMEM((NSEG,), jnp.float32),
        pltpu.VMEM_SHARED((NSEG,), jnp.float32),
    ],
)
def _sc_prep(src_hbm, dst_hbm, typ_hbm, w_hbm, p_hbm, s_hbm,
             zbuf, sbuf0, dbuf, tbuf, wbuf, pbuf, sbuf, segbuf, ones,
             cntl, cnts):
    sid = lax.axis_index("s")
    cid = lax.axis_index("c")

    @pl.loop(0, SEG_SL, step=16)
    def _(i):
        zbuf[pl.ds(i, 16)] = jnp.zeros((16,), jnp.float32)

    @pl.loop(0, SUB, step=16)
    def _(i):
        ones[pl.ds(i, 16)] = jnp.ones((16,), jnp.float32)

    pltpu.sync_copy(zbuf, cnts.at[pl.ds(sid * SEG_SL, SEG_SL)])
    plsc.subcore_barrier()

    @pl.loop(0, EPS, step=SUP)
    def _(i):
        base = sid * EPS + i
        pltpu.sync_copy(dst_hbm.at[pl.ds(base, SUP)], dbuf)
        pltpu.sync_copy(typ_hbm.at[pl.ds(base, SUP)], tbuf)

        @pl.loop(0, ROWS)
        def _(r):
            @pl.loop(0, SUB, step=16)
            def _(j):
                segbuf[r, pl.ds(j, 16)] = (
                    dbuf[pl.ds(r * SUB + j, 16)] * NREL
                    + tbuf[pl.ds(r * SUB + j, 16)])
            pltpu.sync_copy(ones, cnts.at[segbuf.at[r]], add=True)

    plsc.subcore_barrier()
    pltpu.sync_copy(cnts, cntl)

    wid = sid * NC + cid

    @pl.loop(0, EPW, step=SUP)
    def _(i):
        base = wid * EPW + i
        pltpu.sync_copy(src_hbm.at[pl.ds(base, SUP)], sbuf0)
        pltpu.sync_copy(dst_hbm.at[pl.ds(base, SUP)], dbuf)
        pltpu.sync_copy(typ_hbm.at[pl.ds(base, SUP)], tbuf)
        pltpu.sync_copy(w_hbm.at[pl.ds(base, SUP)], wbuf)

        @pl.loop(0, SUP, step=16)
        def _(j):
            sl = pl.ds(j, 16)
            d16 = dbuf[sl]
            t16 = tbuf[sl]
            seg = d16 * NREL + t16
            c = plsc.load_gather(cntl, [seg])
            sbuf[sl] = wbuf[sl] / jnp.maximum(c, 1.0)
            pbuf[sl] = (sbuf0[sl] + (d16 << 14)) + (t16 << 28)

        pltpu.sync_copy(pbuf, p_hbm.at[pl.ds(base, SUP)])
        pltpu.sync_copy(sbuf, s_hbm.at[pl.ds(base, SUP)])


NSUBS = EPW // SUB
ZRR = 25


def _make_agg(d):
    @functools.partial(
        pl.kernel,
        out_type=jax.ShapeDtypeStruct((NC, NN, d), jnp.float32),
        mesh=_mesh,
        compiler_params=_cp,
        scratch_types=[
            pltpu.VMEM((ZRR, d), jnp.float32),
            pltpu.VMEM((EPW,), jnp.int32),
            pltpu.VMEM((EPW,), jnp.float32),
            pltpu.VMEM((2, SUB), jnp.int32),
            pltpu.VMEM((2, SUB), jnp.int32),
            pltpu.VMEM((2, SUB, d), jnp.float32),
            pltpu.VMEM_SHARED((NN, d), jnp.float32),
            pltpu.SemaphoreType.DMA,
            pltpu.SemaphoreType.DMA,
            pltpu.SemaphoreType.DMA,
            pltpu.SemaphoreType.DMA,
        ],
    )
    def _agg(tab_hbm, p_hbm, s_hbm, out_hbm,
             zbuf, pbuf, sb, gidx, didx, rows, acc, gs0, gs1, ss0, ss1):
        sid = lax.axis_index("s")
        cid = lax.axis_index("c")
        wid = sid * NC + cid
        base = wid * EPW
        gsems = (gs0, gs1)
        ssems = (ss0, ss1)

        @pl.loop(0, ZRR)
        def _(r):
            for k in range(d // 16):
                zbuf[r, pl.ds(k * 16, 16)] = jnp.zeros((16,), jnp.float32)

        for k in range(NZR // ZRR):
            pltpu.sync_copy(zbuf, acc.at[pl.ds(sid * NZR + k * ZRR, ZRR)])

        pltpu.sync_copy(p_hbm.at[pl.ds(base, EPW)], pbuf)
        pltpu.sync_copy(s_hbm.at[pl.ds(base, EPW)], sb)
        plsc.subcore_barrier()

        def fill_idx(r, b):
            @pl.loop(0, SUB, step=16)
            def _(j):
                p16 = pbuf[pl.ds(r * SUB + j, 16)]
                gidx[b, pl.ds(j, 16)] = (
                    ((p16 >> 28) & 7) * NN + (p16 & 0x3FFF))
                didx[b, pl.ds(j, 16)] = (p16 >> 14) & 0x3FFF

        def g_desc(b):
            return pltpu.make_async_copy(tab_hbm.at[gidx.at[b]],
                                         rows.at[b], gsems[b])

        def s_desc(b):
            return pltpu.make_async_copy(rows.at[b],
                                         acc.at[didx.at[b]], ssems[b])

        def step(r, b):
            g_desc(b).wait()

            @pl.when(r >= 1)
            def _():
                s_desc(1 - b).wait()

            @pl.when(r < NSUBS - 1)
            def _():
                fill_idx(r + 1, 1 - b)
                g_desc(1 - b).start()

            @pl.loop(0, SUB)
            def _(e):
                spl = plsc.load_gather(sb, [_splat16(r * SUB + e)])
                for k in range(d // 16):
                    sl = pl.ds(k * 16, 16)
                    rows[b, e, sl] = rows[b, e, sl] * spl

            s_desc(b).start(add=True)

        fill_idx(0, 0)
        g_desc(0).start()

        @pl.loop(0, NSUBS)
        def _(r):
            @pl.when(r % 2 == 0)
            def _():
                step(r, 0)

            @pl.when(r % 2 == 1)
            def _():
                step(r, 1)

        s_desc((NSUBS - 1) % 2).wait()
        plsc.subcore_barrier()

        @pl.when(sid == 0)
        def _():
            pltpu.sync_copy(acc, out_hbm.at[cid])

    return _agg


_agg_hid = _make_agg(DHID)
_agg_out = _make_agg(DP)


BN = 400
NB = NN // BN

_DOT = functools.partial(jnp.dot, preferred_element_type=jnp.float32,
                         precision=lax.Precision.HIGHEST)


def _t1_body(x_ref, w_ref, r_ref, b_ref, tab_ref, base_ref):
    xb = x_ref[...]
    for r in range(NREL):
        tab_ref[r] = _DOT(xb, w_ref[r])
    base_ref[...] = _DOT(xb, r_ref[...]) + b_ref[...]


_t1 = pl.pallas_call(
    _t1_body,
    grid=(NB,),
    in_specs=[
        pl.BlockSpec((BN, DIN), lambda i: (i, 0)),
        pl.BlockSpec((NREL, DIN, DHID), lambda i: (0, 0, 0)),
        pl.BlockSpec((DIN, DHID), lambda i: (0, 0)),
        pl.BlockSpec((1, DHID), lambda i: (0, 0)),
    ],
    out_specs=[
        pl.BlockSpec((NREL, BN, DHID), lambda i: (0, i, 0)),
        pl.BlockSpec((BN, DHID), lambda i: (i, 0)),
    ],
    out_shape=[
        jax.ShapeDtypeStruct((NREL, NN, DHID), jnp.float32),
        jax.ShapeDtypeStruct((NN, DHID), jnp.float32),
    ],
)


def _t2_body(b1_ref, a1_ref, w_ref, r_ref, b_ref, tab_ref, base_ref):
    h = jnp.maximum(b1_ref[...] + a1_ref[0] + a1_ref[1], 0.0)
    for r in range(NREL):
        tab_ref[r] = _DOT(h, w_ref[r])
    base_ref[...] = _DOT(h, r_ref[...]) + b_ref[...]


_t2 = pl.pallas_call(
    _t2_body,
    grid=(NB,),
    in_specs=[
        pl.BlockSpec((BN, DHID), lambda i: (i, 0)),
        pl.BlockSpec((NC, BN, DHID), lambda i: (0, i, 0)),
        pl.BlockSpec((NREL, DHID, DP), lambda i: (0, 0, 0)),
        pl.BlockSpec((DHID, DP), lambda i: (0, 0)),
        pl.BlockSpec((1, DP), lambda i: (0, 0)),
    ],
    out_specs=[
        pl.BlockSpec((NREL, BN, DP), lambda i: (0, i, 0)),
        pl.BlockSpec((BN, DP), lambda i: (i, 0)),
    ],
    out_shape=[
        jax.ShapeDtypeStruct((NREL, NN, DP), jnp.float32),
        jax.ShapeDtypeStruct((NN, DP), jnp.float32),
    ],
)


def _t3_body(b2_ref, a2_ref, out_ref):
    z = b2_ref[...] + a2_ref[0] + a2_ref[1]
    mask = lax.broadcasted_iota(jnp.int32, (BN, DP), 1) < DOUT
    zm = jnp.where(mask, z, -1e30)
    m = jnp.max(zm, axis=1, keepdims=True)
    lse = jnp.log(jnp.sum(jnp.exp(zm - m), axis=1, keepdims=True))
    out_ref[...] = z - m - lse


_t3 = pl.pallas_call(
    _t3_body,
    grid=(NB,),
    in_specs=[
        pl.BlockSpec((BN, DP), lambda i: (i, 0)),
        pl.BlockSpec((NC, BN, DP), lambda i: (0, i, 0)),
    ],
    out_specs=pl.BlockSpec((BN, DP), lambda i: (i, 0)),
    out_shape=jax.ShapeDtypeStruct((NN, DP), jnp.float32),
)


@jax.jit
def _run(x, edge_index, edge_weight, edge_type, w1, root1, b1, w2, root2, b2):
    src = edge_index[0]
    dst = edge_index[1]
    et = edge_type

    packed, s = _sc_prep(src, dst, et, edge_weight)
    tab1, base1 = _t1(x, w1, root1, b1.reshape(1, DHID))
    acc1 = _agg_hid(tab1.reshape(NREL * NN, DHID), packed, s)

    w2p = jnp.pad(w2, ((0, 0), (0, 0), (0, DP - DOUT)))
    root2p = jnp.pad(root2, ((0, 0), (0, DP - DOUT)))
    b2p = jnp.pad(b2, (0, DP - DOUT)).reshape(1, DP)
    tab2, base2 = _t2(base1, acc1, w2p, root2p, b2p)
    acc2 = _agg_out(tab2.reshape(NREL * NN, DP), packed, s)

    out = _t3(base2, acc2)
    return out[:, :DOUT]


def kernel(x, edge_index, edge_weight, edge_type, w1, root1, b1,
           w2, root2, b2):
    return _run(x, edge_index, edge_weight, edge_type, w1, root1, b1,
                w2, root2, b2)


# speedup vs baseline: 32.4549x; 1.0207x over previous
import dataclasses
import functools
import jax
import jax.numpy as jnp
from jax import lax
from jax.experimental import pallas as pl
from jax.experimental.pallas import tpu as pltpu
from jax.experimental.pallas import tpu_sc as plsc

NREL, NN, NE = 8, 10000, 320000
DIN, DHID, DOUT = 128, 128, 40
DP = 48
NC, NS = 2, 16
NW = NC * NS
SUB = 80
ROWS = 25
SUP = SUB * ROWS
EPW = NE // NW
EPS = NE // NS
NSEG = NN * NREL
SEG_SL = NSEG // NS
NZR = NN // NS
ZR = 125

_mesh = plsc.VectorSubcoreMesh(core_axis_name="c", subcore_axis_name="s",
                               num_cores=NC, num_subcores=NS)

_cp = pltpu.CompilerParams()
if "needs_layout_passes" in pltpu.CompilerParams.__dataclass_fields__:
    _cp = dataclasses.replace(_cp, needs_layout_passes=False)
if "use_tc_tiling_on_sc" in pltpu.CompilerParams.__dataclass_fields__:
    _cp = dataclasses.replace(_cp, use_tc_tiling_on_sc=False)


def _splat16(v):
    return jnp.broadcast_to(jnp.asarray(v, jnp.int32), (16,))


@functools.partial(
    pl.kernel,
    out_type=(jax.ShapeDtypeStruct((NE,), jnp.int32),
              jax.ShapeDtypeStruct((NE,), jnp.float32)),
    mesh=_mesh,
    compiler_params=_cp,
    scratch_types=[
        pltpu.VMEM((SEG_SL,), jnp.float32),
        pltpu.VMEM((SUP,), jnp.int32),
        pltpu.VMEM((SUP,), jnp.int32),
        pltpu.VMEM((SUP,), jnp.int32),
        pltpu.VMEM((SUP,), jnp.float32),
        pltpu.VMEM((SUP,), jnp.int32),
        pltpu.VMEM((SUP,), jnp.float32),
        pltpu.VMEM((ROWS, SUB), jnp.int32),
        pltpu.VMEM((SUB,), jnp.float32),
        pltpu.VMEM((NSEG,), jnp.float32),
        pltpu.VMEM_SHARED((NSEG,), jnp.float32),
    ],
)
def _sc_prep(src_hbm, dst_hbm, typ_hbm, w_hbm, p_hbm, s_hbm,
             zbuf, sbuf0, dbuf, tbuf, wbuf, pbuf, sbuf, segbuf, ones,
             cntl, cnts):
    sid = lax.axis_index("s")
    cid = lax.axis_index("c")

    @pl.loop(0, SEG_SL, step=16)
    def _(i):
        zbuf[pl.ds(i, 16)] = jnp.zeros((16,), jnp.float32)

    @pl.loop(0, SUB, step=16)
    def _(i):
        ones[pl.ds(i, 16)] = jnp.ones((16,), jnp.float32)

    pltpu.sync_copy(zbuf, cnts.at[pl.ds(sid * SEG_SL, SEG_SL)])
    plsc.subcore_barrier()

    @pl.loop(0, EPS, step=SUP)
    def _(i):
        base = sid * EPS + i
        pltpu.sync_copy(dst_hbm.at[pl.ds(base, SUP)], dbuf)
        pltpu.sync_copy(typ_hbm.at[pl.ds(base, SUP)], tbuf)

        @pl.loop(0, ROWS)
        def _(r):
            @pl.loop(0, SUB, step=16)
            def _(j):
                segbuf[r, pl.ds(j, 16)] = (
                    dbuf[pl.ds(r * SUB + j, 16)] * NREL
                    + tbuf[pl.ds(r * SUB + j, 16)])
            pltpu.sync_copy(ones, cnts.at[segbuf.at[r]], add=True)

    plsc.subcore_barrier()
    pltpu.sync_copy(cnts, cntl)

    wid = sid * NC + cid

    @pl.loop(0, EPW, step=SUP)
    def _(i):
        base = wid * EPW + i
        pltpu.sync_copy(src_hbm.at[pl.ds(base, SUP)], sbuf0)
        pltpu.sync_copy(dst_hbm.at[pl.ds(base, SUP)], dbuf)
        pltpu.sync_copy(typ_hbm.at[pl.ds(base, SUP)], tbuf)
        pltpu.sync_copy(w_hbm.at[pl.ds(base, SUP)], wbuf)

        @pl.loop(0, SUP, step=16)
        def _(j):
            sl = pl.ds(j, 16)
            d16 = dbuf[sl]
            t16 = tbuf[sl]
            seg = d16 * NREL + t16
            c = plsc.load_gather(cntl, [seg])
            sbuf[sl] = wbuf[sl] / jnp.maximum(c, 1.0)
            pbuf[sl] = (sbuf0[sl] + (d16 << 14)) + (t16 << 28)

        pltpu.sync_copy(pbuf, p_hbm.at[pl.ds(base, SUP)])
        pltpu.sync_copy(sbuf, s_hbm.at[pl.ds(base, SUP)])


NSUBS = EPW // SUB
ZRR = 25


def _make_agg(d):
    @functools.partial(
        pl.kernel,
        out_type=jax.ShapeDtypeStruct((NC, NN, d), jnp.float32),
        mesh=_mesh,
        compiler_params=_cp,
        scratch_types=[
            pltpu.VMEM((ZRR, d), jnp.float32),
            pltpu.VMEM((EPW,), jnp.int32),
            pltpu.VMEM((EPW,), jnp.float32),
            pltpu.VMEM((2, SUB), jnp.int32),
            pltpu.VMEM((2, SUB), jnp.int32),
            pltpu.VMEM((2, SUB, d), jnp.float32),
            pltpu.VMEM_SHARED((NN, d), jnp.float32),
            pltpu.SemaphoreType.DMA,
            pltpu.SemaphoreType.DMA,
            pltpu.SemaphoreType.DMA,
            pltpu.SemaphoreType.DMA,
        ],
    )
    def _agg(tab_hbm, p_hbm, s_hbm, out_hbm,
             zbuf, pbuf, sb, gidx, didx, rows, acc, gs0, gs1, ss0, ss1):
        sid = lax.axis_index("s")
        cid = lax.axis_index("c")
        wid = sid * NC + cid
        base = wid * EPW
        gsems = (gs0, gs1)
        ssems = (ss0, ss1)

        @pl.loop(0, ZRR)
        def _(r):
            for k in range(d // 16):
                zbuf[r, pl.ds(k * 16, 16)] = jnp.zeros((16,), jnp.float32)

        for k in range(NZR // ZRR):
            pltpu.sync_copy(zbuf, acc.at[pl.ds(sid * NZR + k * ZRR, ZRR)])

        pltpu.sync_copy(p_hbm.at[pl.ds(base, EPW)], pbuf)
        pltpu.sync_copy(s_hbm.at[pl.ds(base, EPW)], sb)
        plsc.subcore_barrier()

        def fill_idx(r, b):
            @pl.loop(0, SUB, step=16)
            def _(j):
                p16 = pbuf[pl.ds(r * SUB + j, 16)]
                gidx[b, pl.ds(j, 16)] = (
                    ((p16 >> 28) & 7) * NN + (p16 & 0x3FFF))
                didx[b, pl.ds(j, 16)] = (p16 >> 14) & 0x3FFF

        def g_desc(b):
            return pltpu.make_async_copy(tab_hbm.at[gidx.at[b]],
                                         rows.at[b], gsems[b])

        def s_desc(b):
            return pltpu.make_async_copy(rows.at[b],
                                         acc.at[didx.at[b]], ssems[b])

        def step(r, b):
            g_desc(b).wait()

            @pl.when(r >= 1)
            def _():
                s_desc(1 - b).wait()

            @pl.when(r < NSUBS - 1)
            def _():
                fill_idx(r + 1, 1 - b)
                g_desc(1 - b).start()

            @pl.loop(0, SUB, unroll=4)
            def _(e):
                spl = plsc.load_gather(sb, [_splat16(r * SUB + e)])
                for k in range(d // 16):
                    sl = pl.ds(k * 16, 16)
                    rows[b, e, sl] = rows[b, e, sl] * spl

            s_desc(b).start(add=True)

        fill_idx(0, 0)
        g_desc(0).start()

        @pl.loop(0, NSUBS)
        def _(r):
            @pl.when(r % 2 == 0)
            def _():
                step(r, 0)

            @pl.when(r % 2 == 1)
            def _():
                step(r, 1)

        s_desc((NSUBS - 1) % 2).wait()
        plsc.subcore_barrier()

        @pl.when(sid == 0)
        def _():
            pltpu.sync_copy(acc, out_hbm.at[cid])

    return _agg


_agg_hid = _make_agg(DHID)
_agg_out = _make_agg(DP)


BN = 400
NB = NN // BN

_DOT = functools.partial(jnp.dot, preferred_element_type=jnp.float32,
                         precision=lax.Precision.HIGHEST)


def _t1_body(x_ref, w_ref, r_ref, b_ref, tab_ref, base_ref):
    xb = x_ref[...]
    for r in range(NREL):
        tab_ref[r] = _DOT(xb, w_ref[r])
    base_ref[...] = _DOT(xb, r_ref[...]) + b_ref[...]


_t1 = pl.pallas_call(
    _t1_body,
    grid=(NB,),
    in_specs=[
        pl.BlockSpec((BN, DIN), lambda i: (i, 0)),
        pl.BlockSpec((NREL, DIN, DHID), lambda i: (0, 0, 0)),
        pl.BlockSpec((DIN, DHID), lambda i: (0, 0)),
        pl.BlockSpec((1, DHID), lambda i: (0, 0)),
    ],
    out_specs=[
        pl.BlockSpec((NREL, BN, DHID), lambda i: (0, i, 0)),
        pl.BlockSpec((BN, DHID), lambda i: (i, 0)),
    ],
    out_shape=[
        jax.ShapeDtypeStruct((NREL, NN, DHID), jnp.float32),
        jax.ShapeDtypeStruct((NN, DHID), jnp.float32),
    ],
)


def _t2_body(b1_ref, a1_ref, w_ref, r_ref, b_ref, tab_ref, base_ref):
    h = jnp.maximum(b1_ref[...] + a1_ref[0] + a1_ref[1], 0.0)
    for r in range(NREL):
        tab_ref[r] = _DOT(h, w_ref[r])
    base_ref[...] = _DOT(h, r_ref[...]) + b_ref[...]


_t2 = pl.pallas_call(
    _t2_body,
    grid=(NB,),
    in_specs=[
        pl.BlockSpec((BN, DHID), lambda i: (i, 0)),
        pl.BlockSpec((NC, BN, DHID), lambda i: (0, i, 0)),
        pl.BlockSpec((NREL, DHID, DP), lambda i: (0, 0, 0)),
        pl.BlockSpec((DHID, DP), lambda i: (0, 0)),
        pl.BlockSpec((1, DP), lambda i: (0, 0)),
    ],
    out_specs=[
        pl.BlockSpec((NREL, BN, DP), lambda i: (0, i, 0)),
        pl.BlockSpec((BN, DP), lambda i: (i, 0)),
    ],
    out_shape=[
        jax.ShapeDtypeStruct((NREL, NN, DP), jnp.float32),
        jax.ShapeDtypeStruct((NN, DP), jnp.float32),
    ],
)


def _t3_body(b2_ref, a2_ref, out_ref):
    z = b2_ref[...] + a2_ref[0] + a2_ref[1]
    mask = lax.broadcasted_iota(jnp.int32, (BN, DP), 1) < DOUT
    zm = jnp.where(mask, z, -1e30)
    m = jnp.max(zm, axis=1, keepdims=True)
    lse = jnp.log(jnp.sum(jnp.exp(zm - m), axis=1, keepdims=True))
    out_ref[...] = z - m - lse


_t3 = pl.pallas_call(
    _t3_body,
    grid=(NB,),
    in_specs=[
        pl.BlockSpec((BN, DP), lambda i: (i, 0)),
        pl.BlockSpec((NC, BN, DP), lambda i: (0, i, 0)),
    ],
    out_specs=pl.BlockSpec((BN, DP), lambda i: (i, 0)),
    out_shape=jax.ShapeDtypeStruct((NN, DP), jnp.float32),
)


@jax.jit
def _run(x, edge_index, edge_weight, edge_type, w1, root1, b1, w2, root2, b2):
    src = edge_index[0]
    dst = edge_index[1]
    et = edge_type

    packed, s = _sc_prep(src, dst, et, edge_weight)
    tab1, base1 = _t1(x, w1, root1, b1.reshape(1, DHID))
    acc1 = _agg_hid(tab1.reshape(NREL * NN, DHID), packed, s)

    w2p = jnp.pad(w2, ((0, 0), (0, 0), (0, DP - DOUT)))
    root2p = jnp.pad(root2, ((0, 0), (0, DP - DOUT)))
    b2p = jnp.pad(b2, (0, DP - DOUT)).reshape(1, DP)
    tab2, base2 = _t2(base1, acc1, w2p, root2p, b2p)
    acc2 = _agg_out(tab2.reshape(NREL * NN, DP), packed, s)

    out = _t3(base2, acc2)
    return out[:, :DOUT]


def kernel(x, edge_index, edge_weight, edge_type, w1, root1, b1,
           w2, root2, b2):
    return _run(x, edge_index, edge_weight, edge_type, w1, root1, b1,
                w2, root2, b2)


# speedup vs baseline: 35.8431x; 1.1044x over previous
import dataclasses
import functools
import jax
import jax.numpy as jnp
from jax import lax
from jax.experimental import pallas as pl
from jax.experimental.pallas import tpu as pltpu
from jax.experimental.pallas import tpu_sc as plsc

NREL, NN, NE = 8, 10000, 320000
DIN, DHID, DOUT = 128, 128, 40
DP = 48
NC, NS = 2, 16
NW = NC * NS
SUB = 80
ROWS = 25
SUP = SUB * ROWS
EPW = NE // NW
EPS = NE // NS
NSEG = NN * NREL
SEG_SL = NSEG // NS
NZR = NN // NS
ZR = 125

_mesh = plsc.VectorSubcoreMesh(core_axis_name="c", subcore_axis_name="s",
                               num_cores=NC, num_subcores=NS)

_cp = pltpu.CompilerParams()
if "needs_layout_passes" in pltpu.CompilerParams.__dataclass_fields__:
    _cp = dataclasses.replace(_cp, needs_layout_passes=False)
if "use_tc_tiling_on_sc" in pltpu.CompilerParams.__dataclass_fields__:
    _cp = dataclasses.replace(_cp, use_tc_tiling_on_sc=False)


def _splat16(v):
    return jnp.broadcast_to(jnp.asarray(v, jnp.int32), (16,))


@functools.partial(
    pl.kernel,
    out_type=(jax.ShapeDtypeStruct((NE,), jnp.int32),
              jax.ShapeDtypeStruct((NE,), jnp.float32)),
    mesh=_mesh,
    compiler_params=_cp,
    scratch_types=[
        pltpu.VMEM((SEG_SL,), jnp.float32),
        pltpu.VMEM((SUP,), jnp.int32),
        pltpu.VMEM((SUP,), jnp.int32),
        pltpu.VMEM((SUP,), jnp.int32),
        pltpu.VMEM((SUP,), jnp.float32),
        pltpu.VMEM((SUP,), jnp.int32),
        pltpu.VMEM((SUP,), jnp.float32),
        pltpu.VMEM((ROWS, SUB), jnp.int32),
        pltpu.VMEM((SUB,), jnp.float32),
        pltpu.VMEM((NSEG,), jnp.float32),
        pltpu.VMEM_SHARED((NSEG,), jnp.float32),
    ],
)
def _sc_prep(src_hbm, dst_hbm, typ_hbm, w_hbm, p_hbm, s_hbm,
             zbuf, sbuf0, dbuf, tbuf, wbuf, pbuf, sbuf, segbuf, ones,
             cntl, cnts):
    sid = lax.axis_index("s")
    cid = lax.axis_index("c")

    @pl.loop(0, SEG_SL, step=16)
    def _(i):
        zbuf[pl.ds(i, 16)] = jnp.zeros((16,), jnp.float32)

    @pl.loop(0, SUB, step=16)
    def _(i):
        ones[pl.ds(i, 16)] = jnp.ones((16,), jnp.float32)

    pltpu.sync_copy(zbuf, cnts.at[pl.ds(sid * SEG_SL, SEG_SL)])
    plsc.subcore_barrier()

    @pl.loop(0, EPS, step=SUP)
    def _(i):
        base = sid * EPS + i
        pltpu.sync_copy(dst_hbm.at[pl.ds(base, SUP)], dbuf)
        pltpu.sync_copy(typ_hbm.at[pl.ds(base, SUP)], tbuf)

        @pl.loop(0, ROWS)
        def _(r):
            @pl.loop(0, SUB, step=16)
            def _(j):
                segbuf[r, pl.ds(j, 16)] = (
                    dbuf[pl.ds(r * SUB + j, 16)] * NREL
                    + tbuf[pl.ds(r * SUB + j, 16)])
            pltpu.sync_copy(ones, cnts.at[segbuf.at[r]], add=True)

    plsc.subcore_barrier()
    pltpu.sync_copy(cnts, cntl)

    wid = sid * NC + cid

    @pl.loop(0, EPW, step=SUP)
    def _(i):
        base = wid * EPW + i
        pltpu.sync_copy(src_hbm.at[pl.ds(base, SUP)], sbuf0)
        pltpu.sync_copy(dst_hbm.at[pl.ds(base, SUP)], dbuf)
        pltpu.sync_copy(typ_hbm.at[pl.ds(base, SUP)], tbuf)
        pltpu.sync_copy(w_hbm.at[pl.ds(base, SUP)], wbuf)

        @pl.loop(0, SUP, step=16)
        def _(j):
            sl = pl.ds(j, 16)
            d16 = dbuf[sl]
            t16 = tbuf[sl]
            seg = d16 * NREL + t16
            c = plsc.load_gather(cntl, [seg])
            sbuf[sl] = wbuf[sl] / jnp.maximum(c, 1.0)
            pbuf[sl] = (sbuf0[sl] + (d16 << 14)) + (t16 << 28)

        pltpu.sync_copy(pbuf, p_hbm.at[pl.ds(base, SUP)])
        pltpu.sync_copy(sbuf, s_hbm.at[pl.ds(base, SUP)])


NSUBS = EPW // SUB
ZRR = 25


def _make_agg(d):
    @functools.partial(
        pl.kernel,
        out_type=jax.ShapeDtypeStruct((NC, NN, d), jnp.float32),
        mesh=_mesh,
        compiler_params=_cp,
        scratch_types=[
            pltpu.VMEM((ZRR, d), jnp.float32),
            pltpu.VMEM((EPW,), jnp.int32),
            pltpu.VMEM((EPW,), jnp.float32),
            pltpu.VMEM((2, SUB), jnp.int32),
            pltpu.VMEM((2, SUB), jnp.int32),
            pltpu.VMEM((2, SUB, d), jnp.float32),
            pltpu.VMEM_SHARED((NN, d), jnp.float32),
            pltpu.SemaphoreType.DMA,
            pltpu.SemaphoreType.DMA,
            pltpu.SemaphoreType.DMA,
            pltpu.SemaphoreType.DMA,
        ],
    )
    def _agg(tab_hbm, p_hbm, s_hbm, out_hbm,
             zbuf, pbuf, sb, gidx, didx, rows, acc, gs0, gs1, ss0, ss1):
        sid = lax.axis_index("s")
        cid = lax.axis_index("c")
        wid = sid * NC + cid
        base = wid * EPW
        gsems = (gs0, gs1)
        ssems = (ss0, ss1)

        @pl.loop(0, ZRR)
        def _(r):
            for k in range(d // 16):
                zbuf[r, pl.ds(k * 16, 16)] = jnp.zeros((16,), jnp.float32)

        for k in range(NZR // ZRR):
            pltpu.sync_copy(zbuf, acc.at[pl.ds(sid * NZR + k * ZRR, ZRR)])

        pltpu.sync_copy(p_hbm.at[pl.ds(base, EPW)], pbuf)
        pltpu.sync_copy(s_hbm.at[pl.ds(base, EPW)], sb)
        plsc.subcore_barrier()

        def fill_idx(r, b):
            @pl.loop(0, SUB, step=16)
            def _(j):
                p16 = pbuf[pl.ds(r * SUB + j, 16)]
                gidx[b, pl.ds(j, 16)] = (
                    ((p16 >> 28) & 7) * NN + (p16 & 0x3FFF))
                didx[b, pl.ds(j, 16)] = (p16 >> 14) & 0x3FFF

        def g_desc(b):
            return pltpu.make_async_copy(tab_hbm.at[gidx.at[b]],
                                         rows.at[b], gsems[b])

        def s_desc(b):
            return pltpu.make_async_copy(rows.at[b],
                                         acc.at[didx.at[b]], ssems[b])

        def step(r, b):
            g_desc(b).wait()

            @pl.when(r >= 1)
            def _():
                s_desc(1 - b).wait()

            @pl.when(r < NSUBS - 1)
            def _():
                fill_idx(r + 1, 1 - b)
                g_desc(1 - b).start()

            @pl.loop(0, SUB, unroll=4)
            def _(e):
                spl = plsc.load_gather(sb, [_splat16(r * SUB + e)])
                for k in range(d // 16):
                    sl = pl.ds(k * 16, 16)
                    rows[b, e, sl] = rows[b, e, sl] * spl

            s_desc(b).start(add=True)

        fill_idx(0, 0)
        g_desc(0).start()

        @pl.loop(0, NSUBS)
        def _(r):
            @pl.when(r % 2 == 0)
            def _():
                step(r, 0)

            @pl.when(r % 2 == 1)
            def _():
                step(r, 1)

        s_desc((NSUBS - 1) % 2).wait()
        plsc.subcore_barrier()

        @pl.when(sid == 0)
        def _():
            pltpu.sync_copy(acc, out_hbm.at[cid])

    return _agg


_agg_hid = _make_agg(DHID)
_agg_out = _make_agg(DP)


BN = 400
NB = NN // BN

_DOT = functools.partial(jnp.dot, preferred_element_type=jnp.float32)


def _t1_body(x_ref, w_ref, r_ref, b_ref, tab_ref, base_ref):
    xb = x_ref[...]
    for r in range(NREL):
        tab_ref[r] = _DOT(xb, w_ref[r])
    base_ref[...] = _DOT(xb, r_ref[...]) + b_ref[...]


_t1 = pl.pallas_call(
    _t1_body,
    grid=(NB,),
    in_specs=[
        pl.BlockSpec((BN, DIN), lambda i: (i, 0)),
        pl.BlockSpec((NREL, DIN, DHID), lambda i: (0, 0, 0)),
        pl.BlockSpec((DIN, DHID), lambda i: (0, 0)),
        pl.BlockSpec((1, DHID), lambda i: (0, 0)),
    ],
    out_specs=[
        pl.BlockSpec((NREL, BN, DHID), lambda i: (0, i, 0)),
        pl.BlockSpec((BN, DHID), lambda i: (i, 0)),
    ],
    out_shape=[
        jax.ShapeDtypeStruct((NREL, NN, DHID), jnp.float32),
        jax.ShapeDtypeStruct((NN, DHID), jnp.float32),
    ],
)


def _t2_body(b1_ref, a1_ref, w_ref, r_ref, b_ref, tab_ref, base_ref):
    h = jnp.maximum(b1_ref[...] + a1_ref[0] + a1_ref[1], 0.0)
    for r in range(NREL):
        tab_ref[r] = _DOT(h, w_ref[r])
    base_ref[...] = _DOT(h, r_ref[...]) + b_ref[...]


_t2 = pl.pallas_call(
    _t2_body,
    grid=(NB,),
    in_specs=[
        pl.BlockSpec((BN, DHID), lambda i: (i, 0)),
        pl.BlockSpec((NC, BN, DHID), lambda i: (0, i, 0)),
        pl.BlockSpec((NREL, DHID, DP), lambda i: (0, 0, 0)),
        pl.BlockSpec((DHID, DP), lambda i: (0, 0)),
        pl.BlockSpec((1, DP), lambda i: (0, 0)),
    ],
    out_specs=[
        pl.BlockSpec((NREL, BN, DP), lambda i: (0, i, 0)),
        pl.BlockSpec((BN, DP), lambda i: (i, 0)),
    ],
    out_shape=[
        jax.ShapeDtypeStruct((NREL, NN, DP), jnp.float32),
        jax.ShapeDtypeStruct((NN, DP), jnp.float32),
    ],
)


def _t3_body(b2_ref, a2_ref, out_ref):
    z = b2_ref[...] + a2_ref[0] + a2_ref[1]
    mask = lax.broadcasted_iota(jnp.int32, (BN, DP), 1) < DOUT
    zm = jnp.where(mask, z, -1e30)
    m = jnp.max(zm, axis=1, keepdims=True)
    lse = jnp.log(jnp.sum(jnp.exp(zm - m), axis=1, keepdims=True))
    out_ref[...] = z - m - lse


_t3 = pl.pallas_call(
    _t3_body,
    grid=(NB,),
    in_specs=[
        pl.BlockSpec((BN, DP), lambda i: (i, 0)),
        pl.BlockSpec((NC, BN, DP), lambda i: (0, i, 0)),
    ],
    out_specs=pl.BlockSpec((BN, DP), lambda i: (i, 0)),
    out_shape=jax.ShapeDtypeStruct((NN, DP), jnp.float32),
)


@jax.jit
def _run(x, edge_index, edge_weight, edge_type, w1, root1, b1, w2, root2, b2):
    src = edge_index[0]
    dst = edge_index[1]
    et = edge_type

    packed, s = _sc_prep(src, dst, et, edge_weight)
    tab1, base1 = _t1(x, w1, root1, b1.reshape(1, DHID))
    acc1 = _agg_hid(tab1.reshape(NREL * NN, DHID), packed, s)

    w2p = jnp.pad(w2, ((0, 0), (0, 0), (0, DP - DOUT)))
    root2p = jnp.pad(root2, ((0, 0), (0, DP - DOUT)))
    b2p = jnp.pad(b2, (0, DP - DOUT)).reshape(1, DP)
    tab2, base2 = _t2(base1, acc1, w2p, root2p, b2p)
    acc2 = _agg_out(tab2.reshape(NREL * NN, DP), packed, s)

    out = _t3(base2, acc2)
    return out[:, :DOUT]


def kernel(x, edge_index, edge_weight, edge_type, w1, root1, b1,
           w2, root2, b2):
    return _run(x, edge_index, edge_weight, edge_type, w1, root1, b1,
                w2, root2, b2)


# speedup vs baseline: 35.8848x; 1.0012x over previous
import dataclasses
import functools
import jax
import jax.numpy as jnp
from jax import lax
from jax.experimental import pallas as pl
from jax.experimental.pallas import tpu as pltpu
from jax.experimental.pallas import tpu_sc as plsc

NREL, NN, NE = 8, 10000, 320000
DIN, DHID, DOUT = 128, 128, 40
DP = 48
NC, NS = 2, 16
NW = NC * NS
SUB = 80
ROWS = 25
SUP = SUB * ROWS
EPW = NE // NW
EPS = NE // NS
NSEG = NN * NREL
SEG_SL = NSEG // NS
NZR = NN // NS
ZR = 125

_mesh = plsc.VectorSubcoreMesh(core_axis_name="c", subcore_axis_name="s",
                               num_cores=NC, num_subcores=NS)

_cp = pltpu.CompilerParams()
if "needs_layout_passes" in pltpu.CompilerParams.__dataclass_fields__:
    _cp = dataclasses.replace(_cp, needs_layout_passes=False)
if "use_tc_tiling_on_sc" in pltpu.CompilerParams.__dataclass_fields__:
    _cp = dataclasses.replace(_cp, use_tc_tiling_on_sc=False)


def _splat16(v):
    return jnp.broadcast_to(jnp.asarray(v, jnp.int32), (16,))


NSUP_H = EPS // SUP
NSUP_S = EPW // SUP


@functools.partial(
    pl.kernel,
    out_type=(jax.ShapeDtypeStruct((NE,), jnp.int32),
              jax.ShapeDtypeStruct((NE,), jnp.float32)),
    mesh=_mesh,
    compiler_params=_cp,
    scratch_types=[
        pltpu.VMEM((SEG_SL,), jnp.float32),
        pltpu.VMEM((2, SUP), jnp.int32),
        pltpu.VMEM((2, SUP), jnp.int32),
        pltpu.VMEM((SUP,), jnp.int32),
        pltpu.VMEM((SUP,), jnp.float32),
        pltpu.VMEM((SUP,), jnp.int32),
        pltpu.VMEM((SUP,), jnp.float32),
        pltpu.VMEM((2, ROWS, SUB), jnp.int32),
        pltpu.VMEM((SUB,), jnp.float32),
        pltpu.VMEM((NSEG,), jnp.float32),
        pltpu.VMEM_SHARED((NSEG,), jnp.float32),
        pltpu.SemaphoreType.DMA,
        pltpu.SemaphoreType.DMA,
        pltpu.SemaphoreType.DMA,
    ],
)
def _sc_prep(src_hbm, dst_hbm, typ_hbm, w_hbm, p_hbm, s_hbm,
             zbuf, dbuf, tbuf, sbuf0, wbuf, pbuf, sbuf, segbuf, ones,
             cntl, cnts, esem, hsem, osem):
    sid = lax.axis_index("s")
    cid = lax.axis_index("c")

    @pl.loop(0, SEG_SL, step=16)
    def _(i):
        zbuf[pl.ds(i, 16)] = jnp.zeros((16,), jnp.float32)

    @pl.loop(0, SUB, step=16)
    def _(i):
        ones[pl.ds(i, 16)] = jnp.ones((16,), jnp.float32)

    pltpu.sync_copy(zbuf, cnts.at[pl.ds(sid * SEG_SL, SEG_SL)])
    plsc.subcore_barrier()

    hbase = sid * EPS

    def eload(i, h):
        return (pltpu.make_async_copy(
                    dst_hbm.at[pl.ds(hbase + i * SUP, SUP)], dbuf.at[h], esem),
                pltpu.make_async_copy(
                    typ_hbm.at[pl.ds(hbase + i * SUP, SUP)], tbuf.at[h], esem))

    def hscat(h, r):
        return pltpu.make_async_copy(ones, cnts.at[segbuf.at[h, r]], hsem)

    for d in eload(0, 0):
        d.start()
    for i in range(NSUP_H):
        h = i % 2
        for d in eload(i, h):
            d.wait()
        if i + 1 < NSUP_H:
            for d in eload(i + 1, 1 - h):
                d.start()
        if i >= 2:
            @pl.loop(0, ROWS)
            def _(r):
                hscat(h, r).wait()

        @pl.loop(0, ROWS)
        def _(r):
            @pl.loop(0, SUB, step=16)
            def _(j):
                segbuf[h, r, pl.ds(j, 16)] = (
                    dbuf[h, pl.ds(r * SUB + j, 16)] * NREL
                    + tbuf[h, pl.ds(r * SUB + j, 16)])
            hscat(h, r).start(add=True)

    for i in (NSUP_H - 2, NSUP_H - 1):
        @pl.loop(0, ROWS)
        def _(r):
            hscat(i % 2, r).wait()

    plsc.subcore_barrier()
    pltpu.sync_copy(cnts, cntl)

    wid = sid * NC + cid
    sbase = wid * EPW

    def sload(i, h):
        return (pltpu.make_async_copy(
                    dst_hbm.at[pl.ds(sbase + i * SUP, SUP)], dbuf.at[h], esem),
                pltpu.make_async_copy(
                    typ_hbm.at[pl.ds(sbase + i * SUP, SUP)], tbuf.at[h], esem),
                pltpu.make_async_copy(
                    src_hbm.at[pl.ds(sbase + i * SUP, SUP)], sbuf0, esem),
                pltpu.make_async_copy(
                    w_hbm.at[pl.ds(sbase + i * SUP, SUP)], wbuf, esem))

    for i in range(NSUP_S):
        h = i % 2
        if i == 0:
            for d in sload(0, 0):
                d.start()
        for d in sload(i, h):
            d.wait()

        @pl.loop(0, SUP, step=16)
        def _(j):
            sl = pl.ds(j, 16)
            d16 = dbuf[h, sl]
            t16 = tbuf[h, sl]
            seg = d16 * NREL + t16
            c = plsc.load_gather(cntl, [seg])
            sbuf[sl] = wbuf[sl] / jnp.maximum(c, 1.0)
            pbuf[sl] = (sbuf0[sl] + (d16 << 14)) + (t16 << 28)

        pltpu.sync_copy(pbuf, p_hbm.at[pl.ds(sbase + i * SUP, SUP)])
        pltpu.sync_copy(sbuf, s_hbm.at[pl.ds(sbase + i * SUP, SUP)])
        if i + 1 < NSUP_S:
            for d in sload(i + 1, 1 - h):
                d.start()


NSUBS = EPW // SUB
ZRR = 25


def _make_agg(d):
    @functools.partial(
        pl.kernel,
        out_type=jax.ShapeDtypeStruct((NC, NN, d), jnp.float32),
        mesh=_mesh,
        compiler_params=_cp,
        scratch_types=[
            pltpu.VMEM((ZRR, d), jnp.float32),
            pltpu.VMEM((EPW,), jnp.int32),
            pltpu.VMEM((EPW,), jnp.float32),
            pltpu.VMEM((2, SUB), jnp.int32),
            pltpu.VMEM((2, SUB), jnp.int32),
            pltpu.VMEM((2, SUB, d), jnp.float32),
            pltpu.VMEM_SHARED((NN, d), jnp.float32),
            pltpu.SemaphoreType.DMA,
            pltpu.SemaphoreType.DMA,
            pltpu.SemaphoreType.DMA,
            pltpu.SemaphoreType.DMA,
        ],
    )
    def _agg(tab_hbm, p_hbm, s_hbm, out_hbm,
             zbuf, pbuf, sb, gidx, didx, rows, acc, gs0, gs1, ss0, ss1):
        sid = lax.axis_index("s")
        cid = lax.axis_index("c")
        wid = sid * NC + cid
        base = wid * EPW
        gsems = (gs0, gs1)
        ssems = (ss0, ss1)

        @pl.loop(0, ZRR)
        def _(r):
            for k in range(d // 16):
                zbuf[r, pl.ds(k * 16, 16)] = jnp.zeros((16,), jnp.float32)

        for k in range(NZR // ZRR):
            pltpu.sync_copy(zbuf, acc.at[pl.ds(sid * NZR + k * ZRR, ZRR)])

        pltpu.sync_copy(p_hbm.at[pl.ds(base, EPW)], pbuf)
        pltpu.sync_copy(s_hbm.at[pl.ds(base, EPW)], sb)
        plsc.subcore_barrier()

        def fill_idx(r, b):
            @pl.loop(0, SUB, step=16)
            def _(j):
                p16 = pbuf[pl.ds(r * SUB + j, 16)]
                gidx[b, pl.ds(j, 16)] = (
                    ((p16 >> 28) & 7) * NN + (p16 & 0x3FFF))
                didx[b, pl.ds(j, 16)] = (p16 >> 14) & 0x3FFF

        def g_desc(b):
            return pltpu.make_async_copy(tab_hbm.at[gidx.at[b]],
                                         rows.at[b], gsems[b])

        def s_desc(b):
            return pltpu.make_async_copy(rows.at[b],
                                         acc.at[didx.at[b]], ssems[b])

        def step(r, b):
            g_desc(b).wait()

            @pl.when(r >= 1)
            def _():
                s_desc(1 - b).wait()

            @pl.when(r < NSUBS - 1)
            def _():
                fill_idx(r + 1, 1 - b)
                g_desc(1 - b).start()

            @pl.loop(0, SUB, unroll=4)
            def _(e):
                spl = plsc.load_gather(sb, [_splat16(r * SUB + e)])
                for k in range(d // 16):
                    sl = pl.ds(k * 16, 16)
                    rows[b, e, sl] = rows[b, e, sl] * spl

            s_desc(b).start(add=True)

        fill_idx(0, 0)
        g_desc(0).start()

        @pl.loop(0, NSUBS)
        def _(r):
            @pl.when(r % 2 == 0)
            def _():
                step(r, 0)

            @pl.when(r % 2 == 1)
            def _():
                step(r, 1)

        s_desc((NSUBS - 1) % 2).wait()
        plsc.subcore_barrier()

        @pl.when(sid == 0)
        def _():
            pltpu.sync_copy(acc, out_hbm.at[cid])

    return _agg


_agg_hid = _make_agg(DHID)
_agg_out = _make_agg(DP)


BN = 400
NB = NN // BN

_DOT = functools.partial(jnp.dot, preferred_element_type=jnp.float32)


def _t1_body(x_ref, w_ref, r_ref, b_ref, tab_ref, base_ref):
    xb = x_ref[...]
    for r in range(NREL):
        tab_ref[r] = _DOT(xb, w_ref[r])
    base_ref[...] = _DOT(xb, r_ref[...]) + b_ref[...]


_t1 = pl.pallas_call(
    _t1_body,
    grid=(NB,),
    in_specs=[
        pl.BlockSpec((BN, DIN), lambda i: (i, 0)),
        pl.BlockSpec((NREL, DIN, DHID), lambda i: (0, 0, 0)),
        pl.BlockSpec((DIN, DHID), lambda i: (0, 0)),
        pl.BlockSpec((1, DHID), lambda i: (0, 0)),
    ],
    out_specs=[
        pl.BlockSpec((NREL, BN, DHID), lambda i: (0, i, 0)),
        pl.BlockSpec((BN, DHID), lambda i: (i, 0)),
    ],
    out_shape=[
        jax.ShapeDtypeStruct((NREL, NN, DHID), jnp.float32),
        jax.ShapeDtypeStruct((NN, DHID), jnp.float32),
    ],
)


def _t2_body(b1_ref, a1_ref, w_ref, r_ref, b_ref, tab_ref, base_ref):
    h = jnp.maximum(b1_ref[...] + a1_ref[0] + a1_ref[1], 0.0)
    for r in range(NREL):
        tab_ref[r] = _DOT(h, w_ref[r])
    base_ref[...] = _DOT(h, r_ref[...]) + b_ref[...]


_t2 = pl.pallas_call(
    _t2_body,
    grid=(NB,),
    in_specs=[
        pl.BlockSpec((BN, DHID), lambda i: (i, 0)),
        pl.BlockSpec((NC, BN, DHID), lambda i: (0, i, 0)),
        pl.BlockSpec((NREL, DHID, DP), lambda i: (0, 0, 0)),
        pl.BlockSpec((DHID, DP), lambda i: (0, 0)),
        pl.BlockSpec((1, DP), lambda i: (0, 0)),
    ],
    out_specs=[
        pl.BlockSpec((NREL, BN, DP), lambda i: (0, i, 0)),
        pl.BlockSpec((BN, DP), lambda i: (i, 0)),
    ],
    out_shape=[
        jax.ShapeDtypeStruct((NREL, NN, DP), jnp.float32),
        jax.ShapeDtypeStruct((NN, DP), jnp.float32),
    ],
)


def _t3_body(b2_ref, a2_ref, out_ref):
    z = b2_ref[...] + a2_ref[0] + a2_ref[1]
    mask = lax.broadcasted_iota(jnp.int32, (BN, DP), 1) < DOUT
    zm = jnp.where(mask, z, -1e30)
    m = jnp.max(zm, axis=1, keepdims=True)
    lse = jnp.log(jnp.sum(jnp.exp(zm - m), axis=1, keepdims=True))
    out_ref[...] = z - m - lse


_t3 = pl.pallas_call(
    _t3_body,
    grid=(NB,),
    in_specs=[
        pl.BlockSpec((BN, DP), lambda i: (i, 0)),
        pl.BlockSpec((NC, BN, DP), lambda i: (0, i, 0)),
    ],
    out_specs=pl.BlockSpec((BN, DP), lambda i: (i, 0)),
    out_shape=jax.ShapeDtypeStruct((NN, DP), jnp.float32),
)


@jax.jit
def _run(x, edge_index, edge_weight, edge_type, w1, root1, b1, w2, root2, b2):
    src = edge_index[0]
    dst = edge_index[1]
    et = edge_type

    packed, s = _sc_prep(src, dst, et, edge_weight)
    tab1, base1 = _t1(x, w1, root1, b1.reshape(1, DHID))
    acc1 = _agg_hid(tab1.reshape(NREL * NN, DHID), packed, s)

    w2p = jnp.pad(w2, ((0, 0), (0, 0), (0, DP - DOUT)))
    root2p = jnp.pad(root2, ((0, 0), (0, DP - DOUT)))
    b2p = jnp.pad(b2, (0, DP - DOUT)).reshape(1, DP)
    tab2, base2 = _t2(base1, acc1, w2p, root2p, b2p)
    acc2 = _agg_out(tab2.reshape(NREL * NN, DP), packed, s)

    out = _t3(base2, acc2)
    return out[:, :DOUT]


def kernel(x, edge_index, edge_weight, edge_type, w1, root1, b1,
           w2, root2, b2):
    return _run(x, edge_index, edge_weight, edge_type, w1, root1, b1,
                w2, root2, b2)


# speedup vs baseline: 36.8385x; 1.0266x over previous
import dataclasses
import functools
import jax
import jax.numpy as jnp
from jax import lax
from jax.experimental import pallas as pl
from jax.experimental.pallas import tpu as pltpu
from jax.experimental.pallas import tpu_sc as plsc

NREL, NN, NE = 8, 10000, 320000
DIN, DHID, DOUT = 128, 128, 40
DP = 48
NC, NS = 2, 16
NW = NC * NS
SUB = 80
ROWS = 25
SUP = SUB * ROWS
EPW = NE // NW
EPS = NE // NS
NSEG = NN * NREL
SEG_SL = NSEG // NS
NZR = NN // NS
ZR = 125

_mesh = plsc.VectorSubcoreMesh(core_axis_name="c", subcore_axis_name="s",
                               num_cores=NC, num_subcores=NS)

_cp = pltpu.CompilerParams()
if "needs_layout_passes" in pltpu.CompilerParams.__dataclass_fields__:
    _cp = dataclasses.replace(_cp, needs_layout_passes=False)
if "use_tc_tiling_on_sc" in pltpu.CompilerParams.__dataclass_fields__:
    _cp = dataclasses.replace(_cp, use_tc_tiling_on_sc=False)


def _splat16(v):
    return jnp.broadcast_to(jnp.asarray(v, jnp.int32), (16,))


NSUP_H = EPS // SUP
NSUP_S = EPW // SUP


@functools.partial(
    pl.kernel,
    out_type=(jax.ShapeDtypeStruct((NE,), jnp.int32),
              jax.ShapeDtypeStruct((NE,), jnp.float32)),
    mesh=_mesh,
    compiler_params=_cp,
    scratch_types=[
        pltpu.VMEM((SEG_SL,), jnp.float32),
        pltpu.VMEM((2, SUP), jnp.int32),
        pltpu.VMEM((2, SUP), jnp.int32),
        pltpu.VMEM((SUP,), jnp.int32),
        pltpu.VMEM((SUP,), jnp.float32),
        pltpu.VMEM((SUP,), jnp.int32),
        pltpu.VMEM((SUP,), jnp.float32),
        pltpu.VMEM((2, ROWS, SUB), jnp.int32),
        pltpu.VMEM((SUB,), jnp.float32),
        pltpu.VMEM((NSEG,), jnp.float32),
        pltpu.VMEM_SHARED((NSEG,), jnp.float32),
        pltpu.SemaphoreType.DMA,
        pltpu.SemaphoreType.DMA,
        pltpu.SemaphoreType.DMA,
    ],
)
def _sc_prep(src_hbm, dst_hbm, typ_hbm, w_hbm, p_hbm, s_hbm,
             zbuf, dbuf, tbuf, sbuf0, wbuf, pbuf, sbuf, segbuf, ones,
             cntl, cnts, esem, hsem, osem):
    sid = lax.axis_index("s")
    cid = lax.axis_index("c")

    @pl.loop(0, SEG_SL, step=16)
    def _(i):
        zbuf[pl.ds(i, 16)] = jnp.zeros((16,), jnp.float32)

    @pl.loop(0, SUB, step=16)
    def _(i):
        ones[pl.ds(i, 16)] = jnp.ones((16,), jnp.float32)

    pltpu.sync_copy(zbuf, cnts.at[pl.ds(sid * SEG_SL, SEG_SL)])
    plsc.subcore_barrier()

    hbase = sid * EPS

    def eload(i, h):
        return (pltpu.make_async_copy(
                    dst_hbm.at[pl.ds(hbase + i * SUP, SUP)], dbuf.at[h], esem),
                pltpu.make_async_copy(
                    typ_hbm.at[pl.ds(hbase + i * SUP, SUP)], tbuf.at[h], esem))

    def hscat(h, r):
        return pltpu.make_async_copy(ones, cnts.at[segbuf.at[h, r]], hsem)

    for d in eload(0, 0):
        d.start()
    for i in range(NSUP_H):
        h = i % 2
        for d in eload(i, h):
            d.wait()
        if i + 1 < NSUP_H:
            for d in eload(i + 1, 1 - h):
                d.start()
        if i >= 2:
            @pl.loop(0, ROWS)
            def _(r):
                hscat(h, r).wait()

        @pl.loop(0, ROWS)
        def _(r):
            @pl.loop(0, SUB, step=16)
            def _(j):
                segbuf[h, r, pl.ds(j, 16)] = (
                    dbuf[h, pl.ds(r * SUB + j, 16)] * NREL
                    + tbuf[h, pl.ds(r * SUB + j, 16)])
            hscat(h, r).start(add=True)

    for i in (NSUP_H - 2, NSUP_H - 1):
        @pl.loop(0, ROWS)
        def _(r):
            hscat(i % 2, r).wait()

    plsc.subcore_barrier()
    pltpu.sync_copy(cnts, cntl)

    wid = sid * NC + cid
    sbase = wid * EPW

    def sload(i, h):
        return (pltpu.make_async_copy(
                    dst_hbm.at[pl.ds(sbase + i * SUP, SUP)], dbuf.at[h], esem),
                pltpu.make_async_copy(
                    typ_hbm.at[pl.ds(sbase + i * SUP, SUP)], tbuf.at[h], esem),
                pltpu.make_async_copy(
                    src_hbm.at[pl.ds(sbase + i * SUP, SUP)], sbuf0, esem),
                pltpu.make_async_copy(
                    w_hbm.at[pl.ds(sbase + i * SUP, SUP)], wbuf, esem))

    for i in range(NSUP_S):
        h = i % 2
        if i == 0:
            for d in sload(0, 0):
                d.start()
        for d in sload(i, h):
            d.wait()

        @pl.loop(0, SUP, step=16)
        def _(j):
            sl = pl.ds(j, 16)
            d16 = dbuf[h, sl]
            t16 = tbuf[h, sl]
            seg = d16 * NREL + t16
            c = plsc.load_gather(cntl, [seg])
            sbuf[sl] = wbuf[sl] / jnp.maximum(c, 1.0)
            pbuf[sl] = (sbuf0[sl] + (d16 << 14)) + (t16 << 28)

        pltpu.sync_copy(pbuf, p_hbm.at[pl.ds(sbase + i * SUP, SUP)])
        pltpu.sync_copy(sbuf, s_hbm.at[pl.ds(sbase + i * SUP, SUP)])
        if i + 1 < NSUP_S:
            for d in sload(i + 1, 1 - h):
                d.start()


NSUBS = EPW // SUB
ZRR = 25


def _make_agg(d):
    @functools.partial(
        pl.kernel,
        out_type=jax.ShapeDtypeStruct((NC, NN, d), jnp.float32),
        mesh=_mesh,
        compiler_params=_cp,
        scratch_types=[
            pltpu.VMEM((ZRR, d), jnp.float32),
            pltpu.VMEM((3, SUB), jnp.int32),
            pltpu.VMEM((3, SUB), jnp.float32),
            pltpu.VMEM((3, SUB), jnp.int32),
            pltpu.VMEM((3, SUB), jnp.int32),
            pltpu.VMEM((3, SUB, d), jnp.float32),
            pltpu.VMEM_SHARED((NN, d), jnp.float32),
            pltpu.SemaphoreType.DMA,
            pltpu.SemaphoreType.DMA,
            pltpu.SemaphoreType.DMA,
            pltpu.SemaphoreType.DMA,
            pltpu.SemaphoreType.DMA,
            pltpu.SemaphoreType.DMA,
            pltpu.SemaphoreType.DMA,
            pltpu.SemaphoreType.DMA,
            pltpu.SemaphoreType.DMA,
        ],
    )
    def _agg(tab_hbm, p_hbm, s_hbm, out_hbm,
             zbuf, pring, sring, gidx, didx, rows, acc,
             gs0, gs1, gs2, ss0, ss1, ss2, es0, es1, es2):
        sid = lax.axis_index("s")
        cid = lax.axis_index("c")
        wid = sid * NC + cid
        base = wid * EPW
        gsems = (gs0, gs1, gs2)
        ssems = (ss0, ss1, ss2)
        esems = (es0, es1, es2)

        @pl.loop(0, ZRR)
        def _(r):
            for k in range(d // 16):
                zbuf[r, pl.ds(k * 16, 16)] = jnp.zeros((16,), jnp.float32)

        for k in range(NZR // ZRR):
            pltpu.sync_copy(zbuf, acc.at[pl.ds(sid * NZR + k * ZRR, ZRR)])
        plsc.subcore_barrier()

        def e_descs(r, b):
            sl = pl.ds(base + r * SUB, SUB)
            return (pltpu.make_async_copy(p_hbm.at[sl], pring.at[b], esems[b]),
                    pltpu.make_async_copy(s_hbm.at[sl], sring.at[b], esems[b]))

        def fill_idx(b):
            @pl.loop(0, SUB, step=16)
            def _(j):
                p16 = pring[b, pl.ds(j, 16)]
                gidx[b, pl.ds(j, 16)] = (
                    ((p16 >> 28) & 7) * NN + (p16 & 0x3FFF))
                didx[b, pl.ds(j, 16)] = (p16 >> 14) & 0x3FFF

        def g_desc(b):
            return pltpu.make_async_copy(tab_hbm.at[gidx.at[b]],
                                         rows.at[b], gsems[b])

        def s_desc(b):
            return pltpu.make_async_copy(rows.at[b],
                                         acc.at[didx.at[b]], ssems[b])

        def step(r, b):
            g_desc(b).wait()

            @pl.when(r >= 2)
            def _():
                s_desc((b + 1) % 3).wait()

            @pl.when(r < NSUBS - 1)
            def _():
                for dd in e_descs(r + 1, (b + 1) % 3):
                    dd.wait()
                fill_idx((b + 1) % 3)
                g_desc((b + 1) % 3).start()

            @pl.when(r < NSUBS - 2)
            def _():
                for dd in e_descs(r + 2, (b + 2) % 3):
                    dd.start()

            @pl.loop(0, SUB, unroll=4)
            def _(e):
                spl = plsc.load_gather(sring.at[b], [_splat16(e)])
                for k in range(d // 16):
                    sl = pl.ds(k * 16, 16)
                    rows[b, e, sl] = rows[b, e, sl] * spl

            s_desc(b).start(add=True)

        for dd in e_descs(0, 0):
            dd.start()
        for dd in e_descs(1, 1):
            dd.start()
        for dd in e_descs(0, 0):
            dd.wait()
        fill_idx(0)
        g_desc(0).start()

        @pl.loop(0, NSUBS)
        def _(r):
            for c in range(3):
                @pl.when(r % 3 == c)
                def _():
                    step(r, c)

        s_desc((NSUBS - 2) % 3).wait()
        s_desc((NSUBS - 1) % 3).wait()
        plsc.subcore_barrier()

        @pl.when(sid == 0)
        def _():
            pltpu.sync_copy(acc, out_hbm.at[cid])

    return _agg


_agg_hid = _make_agg(DHID)
_agg_out = _make_agg(DP)


BN = 400
NB = NN // BN

_DOT = functools.partial(jnp.dot, preferred_element_type=jnp.float32)


def _t1_body(x_ref, w_ref, r_ref, b_ref, tab_ref, base_ref):
    xb = x_ref[...]
    for r in range(NREL):
        tab_ref[r] = _DOT(xb, w_ref[r])
    base_ref[...] = _DOT(xb, r_ref[...]) + b_ref[...]


_t1 = pl.pallas_call(
    _t1_body,
    grid=(NB,),
    in_specs=[
        pl.BlockSpec((BN, DIN), lambda i: (i, 0)),
        pl.BlockSpec((NREL, DIN, DHID), lambda i: (0, 0, 0)),
        pl.BlockSpec((DIN, DHID), lambda i: (0, 0)),
        pl.BlockSpec((1, DHID), lambda i: (0, 0)),
    ],
    out_specs=[
        pl.BlockSpec((NREL, BN, DHID), lambda i: (0, i, 0)),
        pl.BlockSpec((BN, DHID), lambda i: (i, 0)),
    ],
    out_shape=[
        jax.ShapeDtypeStruct((NREL, NN, DHID), jnp.float32),
        jax.ShapeDtypeStruct((NN, DHID), jnp.float32),
    ],
)


def _t2_body(b1_ref, a1_ref, w_ref, r_ref, b_ref, tab_ref, base_ref):
    h = jnp.maximum(b1_ref[...] + a1_ref[0] + a1_ref[1], 0.0)
    for r in range(NREL):
        tab_ref[r] = _DOT(h, w_ref[r])
    base_ref[...] = _DOT(h, r_ref[...]) + b_ref[...]


_t2 = pl.pallas_call(
    _t2_body,
    grid=(NB,),
    in_specs=[
        pl.BlockSpec((BN, DHID), lambda i: (i, 0)),
        pl.BlockSpec((NC, BN, DHID), lambda i: (0, i, 0)),
        pl.BlockSpec((NREL, DHID, DP), lambda i: (0, 0, 0)),
        pl.BlockSpec((DHID, DP), lambda i: (0, 0)),
        pl.BlockSpec((1, DP), lambda i: (0, 0)),
    ],
    out_specs=[
        pl.BlockSpec((NREL, BN, DP), lambda i: (0, i, 0)),
        pl.BlockSpec((BN, DP), lambda i: (i, 0)),
    ],
    out_shape=[
        jax.ShapeDtypeStruct((NREL, NN, DP), jnp.float32),
        jax.ShapeDtypeStruct((NN, DP), jnp.float32),
    ],
)


def _t3_body(b2_ref, a2_ref, out_ref):
    z = b2_ref[...] + a2_ref[0] + a2_ref[1]
    mask = lax.broadcasted_iota(jnp.int32, (BN, DP), 1) < DOUT
    zm = jnp.where(mask, z, -1e30)
    m = jnp.max(zm, axis=1, keepdims=True)
    lse = jnp.log(jnp.sum(jnp.exp(zm - m), axis=1, keepdims=True))
    out_ref[...] = z - m - lse


_t3 = pl.pallas_call(
    _t3_body,
    grid=(NB,),
    in_specs=[
        pl.BlockSpec((BN, DP), lambda i: (i, 0)),
        pl.BlockSpec((NC, BN, DP), lambda i: (0, i, 0)),
    ],
    out_specs=pl.BlockSpec((BN, DP), lambda i: (i, 0)),
    out_shape=jax.ShapeDtypeStruct((NN, DP), jnp.float32),
)


@jax.jit
def _run(x, edge_index, edge_weight, edge_type, w1, root1, b1, w2, root2, b2):
    src = edge_index[0]
    dst = edge_index[1]
    et = edge_type

    packed, s = _sc_prep(src, dst, et, edge_weight)
    tab1, base1 = _t1(x, w1, root1, b1.reshape(1, DHID))
    acc1 = _agg_hid(tab1.reshape(NREL * NN, DHID), packed, s)

    w2p = jnp.pad(w2, ((0, 0), (0, 0), (0, DP - DOUT)))
    root2p = jnp.pad(root2, ((0, 0), (0, DP - DOUT)))
    b2p = jnp.pad(b2, (0, DP - DOUT)).reshape(1, DP)
    tab2, base2 = _t2(base1, acc1, w2p, root2p, b2p)
    acc2 = _agg_out(tab2.reshape(NREL * NN, DP), packed, s)

    out = _t3(base2, acc2)
    return out[:, :DOUT]


def kernel(x, edge_index, edge_weight, edge_type, w1, root1, b1,
           w2, root2, b2):
    return _run(x, edge_index, edge_weight, edge_type, w1, root1, b1,
                w2, root2, b2)


# speedup vs baseline: 40.3524x; 1.0954x over previous
import dataclasses
import functools
import jax
import jax.numpy as jnp
from jax import lax
from jax.experimental import pallas as pl
from jax.experimental.pallas import tpu as pltpu
from jax.experimental.pallas import tpu_sc as plsc

NREL, NN, NE = 8, 10000, 320000
DIN, DHID, DOUT = 128, 128, 40
DP = 48
NC, NS = 2, 16
NW = NC * NS
SUB = 80
ROWS = 25
SUP = SUB * ROWS
EPW = NE // NW
EPS = NE // NS
NSEG = NN * NREL
SEG_SL = NSEG // NS
NZR = NN // NS
ZR = 125

_mesh = plsc.VectorSubcoreMesh(core_axis_name="c", subcore_axis_name="s",
                               num_cores=NC, num_subcores=NS)

_cp = pltpu.CompilerParams()
if "needs_layout_passes" in pltpu.CompilerParams.__dataclass_fields__:
    _cp = dataclasses.replace(_cp, needs_layout_passes=False)
if "use_tc_tiling_on_sc" in pltpu.CompilerParams.__dataclass_fields__:
    _cp = dataclasses.replace(_cp, use_tc_tiling_on_sc=False)


def _splat16(v):
    return jnp.broadcast_to(jnp.asarray(v, jnp.int32), (16,))


NSUP_H = EPS // SUP
NSUP_S = EPW // SUP


@functools.partial(
    pl.kernel,
    out_type=(jax.ShapeDtypeStruct((NE,), jnp.int32),
              jax.ShapeDtypeStruct((NE,), jnp.float32)),
    mesh=_mesh,
    compiler_params=_cp,
    scratch_types=[
        pltpu.VMEM((SEG_SL,), jnp.float32),
        pltpu.VMEM((2, SUP), jnp.int32),
        pltpu.VMEM((2, SUP), jnp.int32),
        pltpu.VMEM((SUP,), jnp.int32),
        pltpu.VMEM((SUP,), jnp.float32),
        pltpu.VMEM((SUP,), jnp.int32),
        pltpu.VMEM((SUP,), jnp.float32),
        pltpu.VMEM((2, ROWS, SUB), jnp.int32),
        pltpu.VMEM((SUB,), jnp.float32),
        pltpu.VMEM((NSEG,), jnp.float32),
        pltpu.VMEM_SHARED((NSEG,), jnp.float32),
        pltpu.SemaphoreType.DMA,
        pltpu.SemaphoreType.DMA,
        pltpu.SemaphoreType.DMA,
    ],
)
def _sc_prep(src_hbm, dst_hbm, typ_hbm, w_hbm, p_hbm, s_hbm,
             zbuf, dbuf, tbuf, sbuf0, wbuf, pbuf, sbuf, segbuf, ones,
             cntl, cnts, esem, hsem, osem):
    sid = lax.axis_index("s")
    cid = lax.axis_index("c")

    @pl.loop(0, SEG_SL, step=16)
    def _(i):
        zbuf[pl.ds(i, 16)] = jnp.zeros((16,), jnp.float32)

    @pl.loop(0, SUB, step=16)
    def _(i):
        ones[pl.ds(i, 16)] = jnp.ones((16,), jnp.float32)

    pltpu.sync_copy(zbuf, cnts.at[pl.ds(sid * SEG_SL, SEG_SL)])
    plsc.subcore_barrier()

    hbase = sid * EPS

    def eload(i, h):
        return (pltpu.make_async_copy(
                    dst_hbm.at[pl.ds(hbase + i * SUP, SUP)], dbuf.at[h], esem),
                pltpu.make_async_copy(
                    typ_hbm.at[pl.ds(hbase + i * SUP, SUP)], tbuf.at[h], esem))

    def hscat(h, r):
        return pltpu.make_async_copy(ones, cnts.at[segbuf.at[h, r]], hsem)

    for d in eload(0, 0):
        d.start()
    for i in range(NSUP_H):
        h = i % 2
        for d in eload(i, h):
            d.wait()
        if i + 1 < NSUP_H:
            for d in eload(i + 1, 1 - h):
                d.start()
        if i >= 2:
            @pl.loop(0, ROWS)
            def _(r):
                hscat(h, r).wait()

        @pl.loop(0, ROWS)
        def _(r):
            @pl.loop(0, SUB, step=16)
            def _(j):
                segbuf[h, r, pl.ds(j, 16)] = (
                    dbuf[h, pl.ds(r * SUB + j, 16)] * NREL
                    + tbuf[h, pl.ds(r * SUB + j, 16)])
            hscat(h, r).start(add=True)

    for i in (NSUP_H - 2, NSUP_H - 1):
        @pl.loop(0, ROWS)
        def _(r):
            hscat(i % 2, r).wait()

    plsc.subcore_barrier()
    pltpu.sync_copy(cnts, cntl)

    wid = sid * NC + cid
    sbase = wid * EPW

    def sload(i, h):
        return (pltpu.make_async_copy(
                    dst_hbm.at[pl.ds(sbase + i * SUP, SUP)], dbuf.at[h], esem),
                pltpu.make_async_copy(
                    typ_hbm.at[pl.ds(sbase + i * SUP, SUP)], tbuf.at[h], esem),
                pltpu.make_async_copy(
                    src_hbm.at[pl.ds(sbase + i * SUP, SUP)], sbuf0, esem),
                pltpu.make_async_copy(
                    w_hbm.at[pl.ds(sbase + i * SUP, SUP)], wbuf, esem))

    for i in range(NSUP_S):
        h = i % 2
        if i == 0:
            for d in sload(0, 0):
                d.start()
        for d in sload(i, h):
            d.wait()

        @pl.loop(0, SUP, step=16)
        def _(j):
            sl = pl.ds(j, 16)
            d16 = dbuf[h, sl]
            t16 = tbuf[h, sl]
            seg = d16 * NREL + t16
            c = plsc.load_gather(cntl, [seg])
            sbuf[sl] = wbuf[sl] / jnp.maximum(c, 1.0)
            pbuf[sl] = (sbuf0[sl] + (d16 << 14)) + (t16 << 28)

        pltpu.sync_copy(pbuf, p_hbm.at[pl.ds(sbase + i * SUP, SUP)])
        pltpu.sync_copy(sbuf, s_hbm.at[pl.ds(sbase + i * SUP, SUP)])
        if i + 1 < NSUP_S:
            for d in sload(i + 1, 1 - h):
                d.start()


NSUBS = EPW // SUB
ZRR = 25


def _make_agg(d):
    @functools.partial(
        pl.kernel,
        out_type=jax.ShapeDtypeStruct((NC, NN, d), jnp.float32),
        mesh=_mesh,
        compiler_params=_cp,
        scratch_types=[
            pltpu.VMEM((ZRR, d), jnp.float32),
            pltpu.VMEM((3, SUB), jnp.int32),
            pltpu.VMEM((3, SUB), jnp.float32),
            pltpu.VMEM((3, SUB), jnp.int32),
            pltpu.VMEM((3, SUB), jnp.int32),
            pltpu.VMEM((3, SUB, d), jnp.float32),
            pltpu.VMEM_SHARED((NN, d), jnp.float32),
            pltpu.SemaphoreType.DMA,
            pltpu.SemaphoreType.DMA,
            pltpu.SemaphoreType.DMA,
            pltpu.SemaphoreType.DMA,
            pltpu.SemaphoreType.DMA,
            pltpu.SemaphoreType.DMA,
            pltpu.SemaphoreType.DMA,
            pltpu.SemaphoreType.DMA,
            pltpu.SemaphoreType.DMA,
        ],
    )
    def _agg(tab_hbm, p_hbm, s_hbm, out_hbm,
             zbuf, pring, sring, gidx, didx, rows, acc,
             gs0, gs1, gs2, ss0, ss1, ss2, es0, es1, es2):
        sid = lax.axis_index("s")
        cid = lax.axis_index("c")
        wid = sid * NC + cid
        base = wid * EPW
        gsems = (gs0, gs1, gs2)
        ssems = (ss0, ss1, ss2)
        esems = (es0, es1, es2)

        @pl.loop(0, ZRR)
        def _(r):
            for k in range(d // 16):
                zbuf[r, pl.ds(k * 16, 16)] = jnp.zeros((16,), jnp.float32)

        for k in range(NZR // ZRR):
            pltpu.sync_copy(zbuf, acc.at[pl.ds(sid * NZR + k * ZRR, ZRR)])
        plsc.subcore_barrier()

        def e_descs(r, b):
            sl = pl.ds(base + r * SUB, SUB)
            return (pltpu.make_async_copy(p_hbm.at[sl], pring.at[b], esems[b]),
                    pltpu.make_async_copy(s_hbm.at[sl], sring.at[b], esems[b]))

        def fill_idx(b):
            @pl.loop(0, SUB, step=16)
            def _(j):
                p16 = pring[b, pl.ds(j, 16)]
                gidx[b, pl.ds(j, 16)] = (
                    ((p16 >> 28) & 7) * NN + (p16 & 0x3FFF))
                didx[b, pl.ds(j, 16)] = (p16 >> 14) & 0x3FFF

        def g_desc(b):
            return pltpu.make_async_copy(tab_hbm.at[gidx.at[b]],
                                         rows.at[b], gsems[b])

        def s_desc(b):
            return pltpu.make_async_copy(rows.at[b],
                                         acc.at[didx.at[b]], ssems[b])

        def step(r, b):
            @pl.when(r >= 2)
            def _():
                s_desc((b + 1) % 3).wait()

            @pl.when(r < NSUBS - 1)
            def _():
                for dd in e_descs(r + 1, (b + 1) % 3):
                    dd.wait()
                fill_idx((b + 1) % 3)
                g_desc((b + 1) % 3).start()

            @pl.when(r < NSUBS - 2)
            def _():
                for dd in e_descs(r + 2, (b + 2) % 3):
                    dd.start()

            g_desc(b).wait()

            @pl.loop(0, SUB, unroll=4)
            def _(e):
                spl = plsc.load_gather(sring.at[b], [_splat16(e)])
                for k in range(d // 16):
                    sl = pl.ds(k * 16, 16)
                    rows[b, e, sl] = rows[b, e, sl] * spl

            s_desc(b).start(add=True)

        for dd in e_descs(0, 0):
            dd.start()
        for dd in e_descs(1, 1):
            dd.start()
        for dd in e_descs(0, 0):
            dd.wait()
        fill_idx(0)
        g_desc(0).start()

        @pl.loop(0, NSUBS)
        def _(r):
            for c in range(3):
                @pl.when(r % 3 == c)
                def _():
                    step(r, c)

        s_desc((NSUBS - 2) % 3).wait()
        s_desc((NSUBS - 1) % 3).wait()
        plsc.subcore_barrier()

        @pl.when(sid == 0)
        def _():
            pltpu.sync_copy(acc, out_hbm.at[cid])

    return _agg


_agg_hid = _make_agg(DHID)
_agg_out = _make_agg(DP)


BN = 400
NB = NN // BN

_DOT = functools.partial(jnp.dot, preferred_element_type=jnp.float32)


def _t1_body(x_ref, w_ref, r_ref, b_ref, tab_ref, base_ref):
    xb = x_ref[...]
    for r in range(NREL):
        tab_ref[r] = _DOT(xb, w_ref[r])
    base_ref[...] = _DOT(xb, r_ref[...]) + b_ref[...]


_t1 = pl.pallas_call(
    _t1_body,
    grid=(NB,),
    in_specs=[
        pl.BlockSpec((BN, DIN), lambda i: (i, 0)),
        pl.BlockSpec((NREL, DIN, DHID), lambda i: (0, 0, 0)),
        pl.BlockSpec((DIN, DHID), lambda i: (0, 0)),
        pl.BlockSpec((1, DHID), lambda i: (0, 0)),
    ],
    out_specs=[
        pl.BlockSpec((NREL, BN, DHID), lambda i: (0, i, 0)),
        pl.BlockSpec((BN, DHID), lambda i: (i, 0)),
    ],
    out_shape=[
        jax.ShapeDtypeStruct((NREL, NN, DHID), jnp.float32),
        jax.ShapeDtypeStruct((NN, DHID), jnp.float32),
    ],
)


def _t2_body(b1_ref, a1_ref, w_ref, r_ref, b_ref, tab_ref, base_ref):
    h = jnp.maximum(b1_ref[...] + a1_ref[0] + a1_ref[1], 0.0)
    for r in range(NREL):
        tab_ref[r] = _DOT(h, w_ref[r])
    base_ref[...] = _DOT(h, r_ref[...]) + b_ref[...]


_t2 = pl.pallas_call(
    _t2_body,
    grid=(NB,),
    in_specs=[
        pl.BlockSpec((BN, DHID), lambda i: (i, 0)),
        pl.BlockSpec((NC, BN, DHID), lambda i: (0, i, 0)),
        pl.BlockSpec((NREL, DHID, DP), lambda i: (0, 0, 0)),
        pl.BlockSpec((DHID, DP), lambda i: (0, 0)),
        pl.BlockSpec((1, DP), lambda i: (0, 0)),
    ],
    out_specs=[
        pl.BlockSpec((NREL, BN, DP), lambda i: (0, i, 0)),
        pl.BlockSpec((BN, DP), lambda i: (i, 0)),
    ],
    out_shape=[
        jax.ShapeDtypeStruct((NREL, NN, DP), jnp.float32),
        jax.ShapeDtypeStruct((NN, DP), jnp.float32),
    ],
)


def _t3_body(b2_ref, a2_ref, out_ref):
    z = b2_ref[...] + a2_ref[0] + a2_ref[1]
    mask = lax.broadcasted_iota(jnp.int32, (BN, DP), 1) < DOUT
    zm = jnp.where(mask, z, -1e30)
    m = jnp.max(zm, axis=1, keepdims=True)
    lse = jnp.log(jnp.sum(jnp.exp(zm - m), axis=1, keepdims=True))
    out_ref[...] = z - m - lse


_t3 = pl.pallas_call(
    _t3_body,
    grid=(NB,),
    in_specs=[
        pl.BlockSpec((BN, DP), lambda i: (i, 0)),
        pl.BlockSpec((NC, BN, DP), lambda i: (0, i, 0)),
    ],
    out_specs=pl.BlockSpec((BN, DP), lambda i: (i, 0)),
    out_shape=jax.ShapeDtypeStruct((NN, DP), jnp.float32),
)


@jax.jit
def _run(x, edge_index, edge_weight, edge_type, w1, root1, b1, w2, root2, b2):
    src = edge_index[0]
    dst = edge_index[1]
    et = edge_type

    packed, s = _sc_prep(src, dst, et, edge_weight)
    tab1, base1 = _t1(x, w1, root1, b1.reshape(1, DHID))
    acc1 = _agg_hid(tab1.reshape(NREL * NN, DHID), packed, s)

    w2p = jnp.pad(w2, ((0, 0), (0, 0), (0, DP - DOUT)))
    root2p = jnp.pad(root2, ((0, 0), (0, DP - DOUT)))
    b2p = jnp.pad(b2, (0, DP - DOUT)).reshape(1, DP)
    tab2, base2 = _t2(base1, acc1, w2p, root2p, b2p)
    acc2 = _agg_out(tab2.reshape(NREL * NN, DP), packed, s)

    out = _t3(base2, acc2)
    return out[:, :DOUT]


def kernel(x, edge_index, edge_weight, edge_type, w1, root1, b1,
           w2, root2, b2):
    return _run(x, edge_index, edge_weight, edge_type, w1, root1, b1,
                w2, root2, b2)
